# baseline (speedup 1.0000x reference)
import functools

import jax
import jax.numpy as jnp
from jax import lax
from jax.experimental import pallas as pl
from jax.experimental.pallas import tpu as pltpu
from jax.experimental.pallas import tpu_sc as plsc

_MEMORY_SIZE = 50
_SENT_LEN = 20
_DIM = 64
_BATCH = 1024
_VOCAB = 100000
_HALF_SLOTS = _MEMORY_SIZE // 2
_HALF_IDX = _HALF_SLOTS * _SENT_LEN
_HALF_PAD = 512
_TIME_PAD = 56
_ROW_STAGE = 4 * _HALF_PAD + 2 * _TIME_PAD
_TA_OFF = 4 * _HALF_PAD
_TC_OFF = _TA_OFF + _TIME_PAD
_NC = 2
_NS = 16
_NW = _NC * _NS
_ROWS_PER_W = _BATCH // _NW


def _reduce_half(rows_v, t_v, out_u_v, out_t_v, s0):
    lane = lax.iota(jnp.int32, 16)

    @plsc.parallel_loop(0, _HALF_SLOTS, unroll=5)
    def sbody(s):
        base = s * _SENT_LEN
        row_vec = jnp.full((16,), s0 + s, jnp.int32)
        for g in range(_DIM // 32):
            grp = pl.ds(g * 32, 32)
            half = _SENT_LEN // 2
            acc0 = rows_v[base, grp]
            acc1 = rows_v[base + half, grp]
            for t in range(1, half):
                acc0 = acc0 + rows_v[base + t, grp]
                acc1 = acc1 + rows_v[base + half + t, grp]
            acc_e, acc_o = plsc.unpack(
                acc0 + acc1, format=plsc.PackFormat.INTERLEAVED)
            te, to = plsc.unpack(
                t_v[s0 + s, grp], format=plsc.PackFormat.INTERLEAVED)
            col_e = g * 32 + 2 * lane
            col_o = col_e + 1
            plsc.store_scatter(out_u_v, [row_vec, col_e], acc_e)
            plsc.store_scatter(out_u_v, [row_vec, col_o], acc_o)
            plsc.store_scatter(out_t_v, [row_vec, col_e], acc_e + te)
            plsc.store_scatter(out_t_v, [row_vec, col_o], acc_o + to)


def _sc_body(comb, ACT,
             m_out, mu_out, c_out, cu_out,
             idx_v, rows_p, rows_q, ta_v, tc_v, out_u_v, out_t_v,
             semp, semq, semta, semtc):
    wid = lax.axis_index("s") * _NC + lax.axis_index("c")

    def gather(off, buf, sem):
        return [
            pltpu.async_copy(ACT.at[idx_v.at[pl.ds(off + i * 128, 128)]],
                             buf.at[pl.ds(i * 128, 128)], sem)
            for i in range(4)
        ]

    def row_body(bl, carry):
        b = wid * _ROWS_PER_W + bl
        pltpu.sync_copy(comb.at[pl.ds(b * _ROW_STAGE, _ROW_STAGE)], idx_v)
        hta = pltpu.async_copy(
            ACT.at[idx_v.at[pl.ds(_TA_OFF, _TIME_PAD)]], ta_v, semta)
        htc = pltpu.async_copy(
            ACT.at[idx_v.at[pl.ds(_TC_OFF, _TIME_PAD)]], tc_v, semtc)

        ha0 = gather(0, rows_p, semp)
        ha1 = gather(_HALF_PAD, rows_q, semq)
        for h in ha0:
            h.wait()
        hta.wait()
        _reduce_half(rows_p, ta_v, out_u_v, out_t_v, 0)
        hc0 = gather(2 * _HALF_PAD, rows_p, semp)
        for h in ha1:
            h.wait()
        _reduce_half(rows_q, ta_v, out_u_v, out_t_v, _HALF_SLOTS)
        pltpu.sync_copy(out_u_v, mu_out.at[b])
        pltpu.sync_copy(out_t_v, m_out.at[b])
        hc1 = gather(3 * _HALF_PAD, rows_q, semq)
        for h in hc0:
            h.wait()
        htc.wait()
        _reduce_half(rows_p, tc_v, out_u_v, out_t_v, 0)
        for h in hc1:
            h.wait()
        _reduce_half(rows_q, tc_v, out_u_v, out_t_v, _HALF_SLOTS)
        pltpu.sync_copy(out_u_v, cu_out.at[b])
        pltpu.sync_copy(out_t_v, c_out.at[b])
        return carry

    lax.fori_loop(0, _ROWS_PER_W, row_body, 0)


@jax.jit
def _run(comb, ACT):
    out = jax.ShapeDtypeStruct((_BATCH, _MEMORY_SIZE, _DIM), jnp.float32)
    mesh = plsc.VectorSubcoreMesh(core_axis_name="c", subcore_axis_name="s")
    k = functools.partial(
        pl.kernel,
        mesh=mesh,
        out_type=[out, out, out, out],
        compiler_params=pltpu.CompilerParams(use_tc_tiling_on_sc=False,
                                             needs_layout_passes=False),
        scratch_types=[
            pltpu.VMEM((_ROW_STAGE,), jnp.int32),
            pltpu.VMEM((_HALF_PAD, _DIM), jnp.bfloat16),
            pltpu.VMEM((_HALF_PAD, _DIM), jnp.bfloat16),
            pltpu.VMEM((_TIME_PAD, _DIM), jnp.bfloat16),
            pltpu.VMEM((_TIME_PAD, _DIM), jnp.bfloat16),
            pltpu.VMEM((_MEMORY_SIZE, _DIM), jnp.float32),
            pltpu.VMEM((_MEMORY_SIZE, _DIM), jnp.float32),
            pltpu.SemaphoreType.DMA,
            pltpu.SemaphoreType.DMA,
            pltpu.SemaphoreType.DMA,
            pltpu.SemaphoreType.DMA,
        ],
    )(_sc_body)
    return k(comb, ACT)


def kernel(ctx, time, A, C, TA, TC):
    ctx3 = ctx.reshape(_BATCH, 2, _HALF_IDX).astype(jnp.int32)
    ctxp = jnp.pad(ctx3, ((0, 0), (0, 0), (0, _HALF_PAD - _HALF_IDX)),
                   mode="edge").reshape(_BATCH, 2 * _HALF_PAD)
    timep = jnp.pad(time.astype(jnp.int32),
                    ((0, 0), (0, _TIME_PAD - _MEMORY_SIZE)), mode="edge")
    comb = jnp.concatenate(
        [ctxp, ctxp + _VOCAB, timep + 2 * _VOCAB,
         timep + 2 * _VOCAB + _MEMORY_SIZE], axis=1).reshape(-1)
    act = jnp.concatenate([A, C, TA, TC], axis=0).astype(jnp.bfloat16)
    return tuple(_run(comb, act))


# speedup vs baseline: 1.4719x; 1.4719x over previous
import functools

import jax
import jax.numpy as jnp
from jax import lax
from jax.experimental import pallas as pl
from jax.experimental.pallas import tpu as pltpu
from jax.experimental.pallas import tpu_sc as plsc

_MEMORY_SIZE = 50
_SENT_LEN = 20
_DIM = 64
_BATCH = 1024
_HALF_SLOTS = _MEMORY_SIZE // 2
_HALF_IDX = _HALF_SLOTS * _SENT_LEN
_HALF_PAD = 512
_TIME_PAD = 56
_ROW_STAGE = 2 * _HALF_PAD + 2 * _TIME_PAD
_TA_OFF = 2 * _HALF_PAD
_TC_OFF = _TA_OFF + _TIME_PAD
_ROW_OUT = _MEMORY_SIZE * _DIM
_NC = 2
_NS = 16
_NW = _NC * _NS
_ROWS_PER_W = _BATCH // _NW


def _reduce_half(rows_v, t_v, t0, out_u_v, out_t_v, s0):
    lane = lax.iota(jnp.int32, 16)

    @plsc.parallel_loop(0, _HALF_SLOTS, unroll=5)
    def sbody(s):
        base = s * _SENT_LEN
        flat0 = (s0 + s) * _DIM
        for g in range(_DIM // 32):
            grp = pl.ds(g * 32, 32)
            half = _SENT_LEN // 2
            acc0 = rows_v[base, grp]
            acc1 = rows_v[base + half, grp]
            for t in range(1, half):
                acc0 = acc0 + rows_v[base + t, grp]
                acc1 = acc1 + rows_v[base + half + t, grp]
            acc_e, acc_o = plsc.unpack(
                acc0 + acc1, format=plsc.PackFormat.INTERLEAVED)
            te, to = plsc.unpack(
                t_v[t0 + s0 + s, grp], format=plsc.PackFormat.INTERLEAVED)
            col_e = flat0 + g * 32 + 2 * lane
            col_o = col_e + 1
            plsc.store_scatter(out_u_v, [col_e], acc_e)
            plsc.store_scatter(out_u_v, [col_o], acc_o)
            plsc.store_scatter(out_t_v, [col_e], acc_e + te)
            plsc.store_scatter(out_t_v, [col_o], acc_o + to)


def _sc_body(comb, A, C, TT,
             m_out, mu_out, c_out, cu_out,
             idx_v, rows_p, rows_q, t_v, out_u_v, out_t_v,
             semp, semq, semt):
    wid = lax.axis_index("s") * _NC + lax.axis_index("c")

    def gather(table, off, buf, sem):
        return [
            pltpu.async_copy(table.at[idx_v.at[pl.ds(off + i * 128, 128)]],
                             buf.at[pl.ds(i * 128, 128)], sem)
            for i in range(4)
        ]

    def row_body(bl, carry):
        b = wid * _ROWS_PER_W + bl
        pltpu.sync_copy(comb.at[pl.ds(b * _ROW_STAGE, _ROW_STAGE)], idx_v)
        ht = pltpu.async_copy(
            TT.at[idx_v.at[pl.ds(_TA_OFF, 2 * _TIME_PAD)]], t_v, semt)

        ha0 = gather(A, 0, rows_p, semp)
        ha1 = gather(A, _HALF_PAD, rows_q, semq)
        for h in ha0:
            h.wait()
        ht.wait()
        _reduce_half(rows_p, t_v, 0, out_u_v, out_t_v, 0)
        hc0 = gather(C, 0, rows_p, semp)
        for h in ha1:
            h.wait()
        _reduce_half(rows_q, t_v, 0, out_u_v, out_t_v, _HALF_SLOTS)
        pltpu.sync_copy(out_u_v, mu_out.at[pl.ds(b * _ROW_OUT, _ROW_OUT)])
        pltpu.sync_copy(out_t_v, m_out.at[pl.ds(b * _ROW_OUT, _ROW_OUT)])
        hc1 = gather(C, _HALF_PAD, rows_q, semq)
        for h in hc0:
            h.wait()
        _reduce_half(rows_p, t_v, _TIME_PAD, out_u_v, out_t_v, 0)
        for h in hc1:
            h.wait()
        _reduce_half(rows_q, t_v, _TIME_PAD, out_u_v, out_t_v, _HALF_SLOTS)
        pltpu.sync_copy(out_u_v, cu_out.at[pl.ds(b * _ROW_OUT, _ROW_OUT)])
        pltpu.sync_copy(out_t_v, c_out.at[pl.ds(b * _ROW_OUT, _ROW_OUT)])
        return carry

    lax.fori_loop(0, _ROWS_PER_W, row_body, 0)


@jax.jit
def _run(comb, A, C, TT):
    out = jax.ShapeDtypeStruct((_BATCH * _ROW_OUT,), jnp.float32)
    mesh = plsc.VectorSubcoreMesh(core_axis_name="c", subcore_axis_name="s")
    k = functools.partial(
        pl.kernel,
        mesh=mesh,
        out_type=[out, out, out, out],
        compiler_params=pltpu.CompilerParams(use_tc_tiling_on_sc=False,
                                             needs_layout_passes=False),
        scratch_types=[
            pltpu.VMEM((_ROW_STAGE,), jnp.int32),
            pltpu.VMEM((_HALF_PAD, _DIM), jnp.bfloat16),
            pltpu.VMEM((_HALF_PAD, _DIM), jnp.bfloat16),
            pltpu.VMEM((2 * _TIME_PAD, _DIM), jnp.bfloat16),
            pltpu.VMEM((_ROW_OUT,), jnp.float32),
            pltpu.VMEM((_ROW_OUT,), jnp.float32),
            pltpu.SemaphoreType.DMA,
            pltpu.SemaphoreType.DMA,
            pltpu.SemaphoreType.DMA,
        ],
    )(_sc_body)
    return k(comb, A, C, TT)


def kernel(ctx, time, A, C, TA, TC):
    ctx3 = ctx.reshape(_BATCH, 2, _HALF_IDX).astype(jnp.int32)
    ctxp = jnp.pad(ctx3, ((0, 0), (0, 0), (0, _HALF_PAD - _HALF_IDX)),
                   mode="edge").reshape(_BATCH, 2 * _HALF_PAD)
    timep = jnp.pad(time.astype(jnp.int32),
                    ((0, 0), (0, _TIME_PAD - _MEMORY_SIZE)), mode="edge")
    comb = jnp.concatenate(
        [ctxp, timep, timep + _MEMORY_SIZE], axis=1).reshape(-1)
    tt = jnp.concatenate([TA, TC], axis=0).astype(jnp.bfloat16)
    shp = (_BATCH, _MEMORY_SIZE, _DIM)
    m, mu, c, cu = _run(comb, A.astype(jnp.bfloat16), C.astype(jnp.bfloat16),
                        tt)
    return (m.reshape(shp), mu.reshape(shp), c.reshape(shp), cu.reshape(shp))


# speedup vs baseline: 1.5183x; 1.0315x over previous
import functools

import jax
import jax.numpy as jnp
from jax import lax
from jax.experimental import pallas as pl
from jax.experimental.pallas import tpu as pltpu
from jax.experimental.pallas import tpu_sc as plsc

_MEMORY_SIZE = 50
_SENT_LEN = 20
_DIM = 64
_BATCH = 1024
_HALF_SLOTS = _MEMORY_SIZE // 2
_HALF_IDX = _HALF_SLOTS * _SENT_LEN
_HALF_PAD = 512
_TIME_PAD = 56
_ROW_STAGE = 2 * _HALF_PAD + 2 * _TIME_PAD
_TA_OFF = 2 * _HALF_PAD
_TC_OFF = _TA_OFF + _TIME_PAD
_ROW_OUT = _MEMORY_SIZE * _DIM
_NC = 2
_NS = 16
_NW = _NC * _NS
_ROWS_PER_W = _BATCH // _NW


def _reduce_half(rows_v, t_v, t0, out_u_v, out_t_v, s0):
    lane = lax.iota(jnp.int32, 16)

    @plsc.parallel_loop(0, _HALF_SLOTS, unroll=5)
    def sbody(s):
        base = s * _SENT_LEN
        flat0 = (s0 + s) * _DIM
        for g in range(_DIM // 32):
            grp = pl.ds(g * 32, 32)
            half = _SENT_LEN // 2
            acc0 = rows_v[base, grp]
            acc1 = rows_v[base + half, grp]
            for t in range(1, half):
                acc0 = acc0 + rows_v[base + t, grp]
                acc1 = acc1 + rows_v[base + half + t, grp]
            acc_e, acc_o = plsc.unpack(
                acc0 + acc1, format=plsc.PackFormat.INTERLEAVED)
            te, to = plsc.unpack(
                t_v[t0 + s0 + s, grp], format=plsc.PackFormat.INTERLEAVED)
            col_e = flat0 + g * 32 + 2 * lane
            col_o = col_e + 1
            plsc.store_scatter(out_u_v, [col_e], acc_e)
            plsc.store_scatter(out_u_v, [col_o], acc_o)
            plsc.store_scatter(out_t_v, [col_e], acc_e + te)
            plsc.store_scatter(out_t_v, [col_o], acc_o + to)


def _sc_body(comb, A, C, TT,
             m_out, mu_out, c_out, cu_out,
             idx0_v, idx1_v, rows_p, rows_q, t0_v, t1_v,
             au_v, at_v, cu_v, ct_v,
             semp, semq, semt, semsta, semstc):
    wid = lax.axis_index("s") * _NC + lax.axis_index("c")

    def gather(table, idx_v, off, buf, sem):
        return [
            pltpu.async_copy(table.at[idx_v.at[pl.ds(off + i * 128, 128)]],
                             buf.at[pl.ds(i * 128, 128)], sem)
            for i in range(4)
        ]

    def pair_body(g, carry):
        b0 = wid * _ROWS_PER_W + 2 * g
        b1 = b0 + 1
        pltpu.sync_copy(comb.at[pl.ds(b0 * _ROW_STAGE, _ROW_STAGE)], idx0_v)
        ha00 = gather(A, idx0_v, 0, rows_p, semp)
        ha01 = gather(A, idx0_v, _HALF_PAD, rows_q, semq)
        ht0 = pltpu.async_copy(
            TT.at[idx0_v.at[pl.ds(_TA_OFF, 2 * _TIME_PAD)]], t0_v, semt)
        pltpu.sync_copy(comb.at[pl.ds(b1 * _ROW_STAGE, _ROW_STAGE)], idx1_v)
        ht1 = pltpu.async_copy(
            TT.at[idx1_v.at[pl.ds(_TA_OFF, 2 * _TIME_PAD)]], t1_v, semt)

        for h in ha00:
            h.wait()
        ht0.wait()
        _reduce_half(rows_p, t0_v, 0, au_v, at_v, 0)
        hc00 = gather(C, idx0_v, 0, rows_p, semp)
        for h in ha01:
            h.wait()
        _reduce_half(rows_q, t0_v, 0, au_v, at_v, _HALF_SLOTS)
        sta = [pltpu.async_copy(
                   au_v, mu_out.at[pl.ds(b0 * _ROW_OUT, _ROW_OUT)], semsta),
               pltpu.async_copy(
                   at_v, m_out.at[pl.ds(b0 * _ROW_OUT, _ROW_OUT)], semsta)]
        hc01 = gather(C, idx0_v, _HALF_PAD, rows_q, semq)
        for h in hc00:
            h.wait()
        _reduce_half(rows_p, t0_v, _TIME_PAD, cu_v, ct_v, 0)
        ha10 = gather(A, idx1_v, 0, rows_p, semp)
        for h in hc01:
            h.wait()
        _reduce_half(rows_q, t0_v, _TIME_PAD, cu_v, ct_v, _HALF_SLOTS)
        stc = [pltpu.async_copy(
                   cu_v, cu_out.at[pl.ds(b0 * _ROW_OUT, _ROW_OUT)], semstc),
               pltpu.async_copy(
                   ct_v, c_out.at[pl.ds(b0 * _ROW_OUT, _ROW_OUT)], semstc)]
        ha11 = gather(A, idx1_v, _HALF_PAD, rows_q, semq)
        for h in ha10:
            h.wait()
        ht1.wait()
        for h in sta:
            h.wait()
        _reduce_half(rows_p, t1_v, 0, au_v, at_v, 0)
        hc10 = gather(C, idx1_v, 0, rows_p, semp)
        for h in ha11:
            h.wait()
        _reduce_half(rows_q, t1_v, 0, au_v, at_v, _HALF_SLOTS)
        sta2 = [pltpu.async_copy(
                    au_v, mu_out.at[pl.ds(b1 * _ROW_OUT, _ROW_OUT)], semsta),
                pltpu.async_copy(
                    at_v, m_out.at[pl.ds(b1 * _ROW_OUT, _ROW_OUT)], semsta)]
        hc11 = gather(C, idx1_v, _HALF_PAD, rows_q, semq)
        for h in hc10:
            h.wait()
        for h in stc:
            h.wait()
        _reduce_half(rows_p, t1_v, _TIME_PAD, cu_v, ct_v, 0)
        for h in hc11:
            h.wait()
        _reduce_half(rows_q, t1_v, _TIME_PAD, cu_v, ct_v, _HALF_SLOTS)
        for h in sta2:
            h.wait()
        pltpu.sync_copy(cu_v, cu_out.at[pl.ds(b1 * _ROW_OUT, _ROW_OUT)])
        pltpu.sync_copy(ct_v, c_out.at[pl.ds(b1 * _ROW_OUT, _ROW_OUT)])
        return carry

    lax.fori_loop(0, _ROWS_PER_W // 2, pair_body, 0)


@jax.jit
def _run(comb, A, C, TT):
    out = jax.ShapeDtypeStruct((_BATCH * _ROW_OUT,), jnp.float32)
    mesh = plsc.VectorSubcoreMesh(core_axis_name="c", subcore_axis_name="s")
    k = functools.partial(
        pl.kernel,
        mesh=mesh,
        out_type=[out, out, out, out],
        compiler_params=pltpu.CompilerParams(use_tc_tiling_on_sc=False,
                                             needs_layout_passes=False),
        scratch_types=[
            pltpu.VMEM((_ROW_STAGE,), jnp.int32),
            pltpu.VMEM((_ROW_STAGE,), jnp.int32),
            pltpu.VMEM((_HALF_PAD, _DIM), jnp.bfloat16),
            pltpu.VMEM((_HALF_PAD, _DIM), jnp.bfloat16),
            pltpu.VMEM((2 * _TIME_PAD, _DIM), jnp.bfloat16),
            pltpu.VMEM((2 * _TIME_PAD, _DIM), jnp.bfloat16),
            pltpu.VMEM((_ROW_OUT,), jnp.float32),
            pltpu.VMEM((_ROW_OUT,), jnp.float32),
            pltpu.VMEM((_ROW_OUT,), jnp.float32),
            pltpu.VMEM((_ROW_OUT,), jnp.float32),
            pltpu.SemaphoreType.DMA,
            pltpu.SemaphoreType.DMA,
            pltpu.SemaphoreType.DMA,
            pltpu.SemaphoreType.DMA,
            pltpu.SemaphoreType.DMA,
        ],
    )(_sc_body)
    return k(comb, A, C, TT)


def kernel(ctx, time, A, C, TA, TC):
    ctx3 = ctx.reshape(_BATCH, 2, _HALF_IDX).astype(jnp.int32)
    ctxp = jnp.pad(ctx3, ((0, 0), (0, 0), (0, _HALF_PAD - _HALF_IDX)),
                   mode="edge").reshape(_BATCH, 2 * _HALF_PAD)
    timep = jnp.pad(time.astype(jnp.int32),
                    ((0, 0), (0, _TIME_PAD - _MEMORY_SIZE)), mode="edge")
    comb = jnp.concatenate(
        [ctxp, timep, timep + _MEMORY_SIZE], axis=1).reshape(-1)
    tt = jnp.concatenate([TA, TC], axis=0).astype(jnp.bfloat16)
    shp = (_BATCH, _MEMORY_SIZE, _DIM)
    m, mu, c, cu = _run(comb, A.astype(jnp.bfloat16), C.astype(jnp.bfloat16),
                        tt)
    return (m.reshape(shp), mu.reshape(shp), c.reshape(shp), cu.reshape(shp))


# speedup vs baseline: 1.5607x; 1.0279x over previous
import functools

import jax
import jax.numpy as jnp
from jax import lax
from jax.experimental import pallas as pl
from jax.experimental.pallas import tpu as pltpu
from jax.experimental.pallas import tpu_sc as plsc

_MEMORY_SIZE = 50
_SENT_LEN = 20
_DIM = 64
_BATCH = 1024
_HALF_SLOTS = _MEMORY_SIZE // 2
_HALF_IDX = _HALF_SLOTS * _SENT_LEN
_HALF_PAD = 512
_TIME_PAD = 56
_ROW_STAGE = 2 * _HALF_PAD + 2 * _TIME_PAD
_TA_OFF = 2 * _HALF_PAD
_TC_OFF = _TA_OFF + _TIME_PAD
_ROW_OUT = _MEMORY_SIZE * _DIM
_NC = 2
_NS = 16
_NW = _NC * _NS
_ROWS_PER_W = _BATCH // _NW


def _reduce_half(rows_v, t_v, t0, out_u_v, out_t_v, s0):
    lane = lax.iota(jnp.int32, 16)

    @plsc.parallel_loop(0, _HALF_SLOTS, unroll=5)
    def sbody(s):
        base = s * _SENT_LEN
        flat0 = (s0 + s) * _DIM
        for g in range(_DIM // 32):
            grp = pl.ds(g * 32, 32)
            half = _SENT_LEN // 2
            acc0 = rows_v[base, grp]
            acc1 = rows_v[base + half, grp]
            for t in range(1, half):
                acc0 = acc0 + rows_v[base + t, grp]
                acc1 = acc1 + rows_v[base + half + t, grp]
            acc_e, acc_o = plsc.unpack(
                acc0 + acc1, format=plsc.PackFormat.INTERLEAVED)
            te, to = plsc.unpack(
                t_v[t0 + s0 + s, grp], format=plsc.PackFormat.INTERLEAVED)
            col_e = flat0 + g * 32 + 2 * lane
            col_o = col_e + 1
            plsc.store_scatter(out_u_v, [col_e], acc_e)
            plsc.store_scatter(out_u_v, [col_o], acc_o)
            plsc.store_scatter(out_t_v, [col_e], acc_e + te)
            plsc.store_scatter(out_t_v, [col_o], acc_o + to)


def _sc_body(comb, A, C, TT,
             m_out, mu_out, c_out, cu_out,
             idx0_v, idx1_v, rows_p, rows_q, rows_r, t0_v, t1_v,
             au_v, at_v, cu_v, ct_v,
             semp, semq, semr, semt, semt2, semsta, semstc):
    wid = lax.axis_index("s") * _NC + lax.axis_index("c")

    def gather(table, idx_v, off, buf, sem):
        return [
            pltpu.async_copy(table.at[idx_v.at[pl.ds(off + i * 128, 128)]],
                             buf.at[pl.ds(i * 128, 128)], sem)
            for i in range(4)
        ]

    rows = [rows_p, rows_q, rows_r]
    sems = [semp, semq, semr]

    def pair_body(g, carry):
        b0 = wid * _ROWS_PER_W + 2 * g
        b1 = b0 + 1
        pltpu.sync_copy(comb.at[pl.ds(b0 * _ROW_STAGE, _ROW_STAGE)], idx0_v)
        ht0 = pltpu.async_copy(
            TT.at[idx0_v.at[pl.ds(_TA_OFF, 2 * _TIME_PAD)]], t0_v, semt)
        pltpu.sync_copy(comb.at[pl.ds(b1 * _ROW_STAGE, _ROW_STAGE)], idx1_v)
        ht1 = pltpu.async_copy(
            TT.at[idx1_v.at[pl.ds(_TA_OFF, 2 * _TIME_PAD)]], t1_v, semt2)

        chunks = []
        for bb, idx, tv in ((b0, idx0_v, t0_v), (b1, idx1_v, t1_v)):
            for table, toff, uo, to in ((A, 0, mu_out, m_out),
                                        (C, _TIME_PAD, cu_out, c_out)):
                for h in range(2):
                    chunks.append((table, idx, h * _HALF_PAD, tv, toff,
                                   uo, to, h * _HALF_SLOTS, bb))

        hs = [None] * 8
        for k in range(3):
            t, idx, off, _, _, _, _, _, _ = chunks[k]
            hs[k] = gather(t, idx, off, rows[k % 3], sems[k % 3])
        stores = []
        for k in range(8):
            _, _, _, tv, toff, uo, to, s0, bb = chunks[k]
            for h in hs[k]:
                h.wait()
            if k == 0:
                ht0.wait()
            if k == 4:
                ht1.wait()
            even = (k % 4) < 2
            ou, ot = (au_v, at_v) if even else (cu_v, ct_v)
            if k in (2, 3, 6, 7) and stores:
                for h in stores.pop(0):
                    h.wait()
            _reduce_half(rows[k % 3], tv, toff, ou, ot, s0)
            if k + 3 < 8:
                t2, idx2, off2, _, _, _, _, _, _ = chunks[k + 3]
                hs[k + 3] = gather(t2, idx2, off2, rows[k % 3], sems[k % 3])
            if k % 2 == 1:
                sem = semsta if even else semstc
                stores.append([
                    pltpu.async_copy(
                        ou, uo.at[pl.ds(bb * _ROW_OUT, _ROW_OUT)], sem),
                    pltpu.async_copy(
                        ot, to.at[pl.ds(bb * _ROW_OUT, _ROW_OUT)], sem)])
        for st in stores:
            for h in st:
                h.wait()
        return carry

    lax.fori_loop(0, _ROWS_PER_W // 2, pair_body, 0)


@jax.jit
def _run(comb, A, C, TT):
    out = jax.ShapeDtypeStruct((_BATCH * _ROW_OUT,), jnp.float32)
    mesh = plsc.VectorSubcoreMesh(core_axis_name="c", subcore_axis_name="s")
    k = functools.partial(
        pl.kernel,
        mesh=mesh,
        out_type=[out, out, out, out],
        compiler_params=pltpu.CompilerParams(use_tc_tiling_on_sc=False,
                                             needs_layout_passes=False),
        scratch_types=[
            pltpu.VMEM((_ROW_STAGE,), jnp.int32),
            pltpu.VMEM((_ROW_STAGE,), jnp.int32),
            pltpu.VMEM((_HALF_PAD, _DIM), jnp.bfloat16),
            pltpu.VMEM((_HALF_PAD, _DIM), jnp.bfloat16),
            pltpu.VMEM((_HALF_PAD, _DIM), jnp.bfloat16),
            pltpu.VMEM((2 * _TIME_PAD, _DIM), jnp.bfloat16),
            pltpu.VMEM((2 * _TIME_PAD, _DIM), jnp.bfloat16),
            pltpu.VMEM((_ROW_OUT,), jnp.float32),
            pltpu.VMEM((_ROW_OUT,), jnp.float32),
            pltpu.VMEM((_ROW_OUT,), jnp.float32),
            pltpu.VMEM((_ROW_OUT,), jnp.float32),
            pltpu.SemaphoreType.DMA,
            pltpu.SemaphoreType.DMA,
            pltpu.SemaphoreType.DMA,
            pltpu.SemaphoreType.DMA,
            pltpu.SemaphoreType.DMA,
            pltpu.SemaphoreType.DMA,
            pltpu.SemaphoreType.DMA,
        ],
    )(_sc_body)
    return k(comb, A, C, TT)


def kernel(ctx, time, A, C, TA, TC):
    ctx3 = ctx.reshape(_BATCH, 2, _HALF_IDX).astype(jnp.int32)
    ctxp = jnp.pad(ctx3, ((0, 0), (0, 0), (0, _HALF_PAD - _HALF_IDX)),
                   mode="edge").reshape(_BATCH, 2 * _HALF_PAD)
    timep = jnp.pad(time.astype(jnp.int32),
                    ((0, 0), (0, _TIME_PAD - _MEMORY_SIZE)), mode="edge")
    comb = jnp.concatenate(
        [ctxp, timep, timep + _MEMORY_SIZE], axis=1).reshape(-1)
    tt = jnp.concatenate([TA, TC], axis=0).astype(jnp.bfloat16)
    shp = (_BATCH, _MEMORY_SIZE, _DIM)
    m, mu, c, cu = _run(comb, A.astype(jnp.bfloat16), C.astype(jnp.bfloat16),
                        tt)
    return (m.reshape(shp), mu.reshape(shp), c.reshape(shp), cu.reshape(shp))
